# baseline (speedup 1.0000x reference)
import functools

import jax
import jax.numpy as jnp
from jax import lax
from jax.experimental import pallas as pl
from jax.experimental.pallas import tpu as pltpu
from jax.experimental.pallas import tpu_sc as plsc

NUM_EMBEDDINGS = 1000000
EMBEDDING_DIM = 64
BATCH = 16384
FIELDS = 26

NC = 2
NS = 16
NW = NC * NS

B_TOTAL = BATCH * FIELDS
B_PER_W = B_TOTAL // NW
CHUNK = 128
CHUNKS_PER_W = B_PER_W // CHUNK


def _gather_body(table, idx, out, idx_v, buf0, buf1, sem0, sem1):
    cid = lax.axis_index("c")
    sid = lax.axis_index("s")
    wid = sid * NC + cid
    row0 = wid * B_PER_W

    pltpu.sync_copy(idx.at[pl.ds(wid * CHUNKS_PER_W, CHUNKS_PER_W)], idx_v)

    bufs = (buf0, buf1)
    sems = (sem0, sem1)

    def start(c, b):
        pltpu.async_copy(table.at[idx_v.at[c]], bufs[b], sems[b])

    def finish(c, b):
        pltpu.make_async_copy(table.at[idx_v.at[c]], bufs[b], sems[b]).wait()
        pltpu.sync_copy(bufs[b], out.at[pl.ds(row0 + c * CHUNK, CHUNK)])

    start(0, 0)
    start(1, 1)

    @pl.loop(0, CHUNKS_PER_W - 2, step=2)
    def _(c0):
        for b in range(2):
            c = c0 + b
            finish(c, b)
            start(c + 2, b)

    for b in range(2):
        finish(CHUNKS_PER_W - 2 + b, b)


@jax.jit
def _embedding_gather(x_flat, embeddings):
    mesh = plsc.VectorSubcoreMesh(core_axis_name="c", subcore_axis_name="s")
    k = functools.partial(
        pl.kernel,
        mesh=mesh,
        out_type=jax.ShapeDtypeStruct((B_TOTAL, EMBEDDING_DIM), jnp.float32),
        scratch_types=[
            pltpu.VMEM((CHUNKS_PER_W, CHUNK), jnp.int32),
            pltpu.VMEM((CHUNK, EMBEDDING_DIM), jnp.float32),
            pltpu.VMEM((CHUNK, EMBEDDING_DIM), jnp.float32),
            pltpu.SemaphoreType.DMA,
            pltpu.SemaphoreType.DMA,
        ],
        compiler_params=pltpu.CompilerParams(use_tc_tiling_on_sc=False),
    )(_gather_body)
    idx2d = x_flat.reshape(B_TOTAL // CHUNK, CHUNK)
    return k(embeddings, idx2d)


def kernel(x, embeddings):
    out = _embedding_gather(x.reshape(-1), embeddings)
    return out.reshape(BATCH, FIELDS, EMBEDDING_DIM)


# speedup vs baseline: 1.0146x; 1.0146x over previous
import functools

import jax
import jax.numpy as jnp
from jax import lax
from jax.experimental import pallas as pl
from jax.experimental.pallas import tpu as pltpu
from jax.experimental.pallas import tpu_sc as plsc

NUM_EMBEDDINGS = 1000000
EMBEDDING_DIM = 64
BATCH = 16384
FIELDS = 26

NC = 2
NS = 16
NW = NC * NS

B_TOTAL = BATCH * FIELDS
B_PER_W = B_TOTAL // NW
CHUNK = 128
CHUNKS_PER_W = B_PER_W // CHUNK


NBUF = 4
LAG = 2


def _gather_body(table, idx, out, idx_v, bufs_v, gsem, wsem):
    cid = lax.axis_index("c")
    sid = lax.axis_index("s")
    wid = sid * NC + cid
    row0 = wid * B_PER_W

    pltpu.sync_copy(idx.at[pl.ds(wid * CHUNKS_PER_W, CHUNKS_PER_W)], idx_v)

    def start_gather(c, b):
        pltpu.async_copy(table.at[idx_v.at[c]], bufs_v.at[b], gsem.at[b])

    def wait_gather(c, b):
        pltpu.make_async_copy(table.at[idx_v.at[c]], bufs_v.at[b], gsem.at[b]).wait()

    def start_write(c, b):
        pltpu.async_copy(bufs_v.at[b], out.at[pl.ds(row0 + c * CHUNK, CHUNK)], wsem.at[b])

    def wait_write(c, b):
        pltpu.make_async_copy(
            bufs_v.at[b], out.at[pl.ds(row0 + c * CHUNK, CHUNK)], wsem.at[b]
        ).wait()

    for c in range(NBUF):
        start_gather(c, c)
    for c in range(LAG):
        wait_gather(c, c)
        start_write(c, c)

    @pl.loop(NBUF, CHUNKS_PER_W, step=NBUF)
    def _(c0):
        for b in range(NBUF):
            c = c0 + b
            wait_write(c - NBUF, b)
            start_gather(c, b)
            cw = c - NBUF + LAG
            bw = (cw) % NBUF
            wait_gather(cw, bw)
            start_write(cw, bw)

    for c in range(CHUNKS_PER_W - NBUF + LAG, CHUNKS_PER_W):
        b = c % NBUF
        wait_gather(c, b)
        start_write(c, b)
    for c in range(CHUNKS_PER_W - NBUF, CHUNKS_PER_W):
        b = c % NBUF
        wait_write(c, b)


@jax.jit
def _embedding_gather(x_flat, embeddings):
    mesh = plsc.VectorSubcoreMesh(core_axis_name="c", subcore_axis_name="s")
    k = functools.partial(
        pl.kernel,
        mesh=mesh,
        out_type=jax.ShapeDtypeStruct((B_TOTAL, EMBEDDING_DIM), jnp.float32),
        scratch_types=[
            pltpu.VMEM((CHUNKS_PER_W, CHUNK), jnp.int32),
            pltpu.VMEM((NBUF, CHUNK, EMBEDDING_DIM), jnp.float32),
            pltpu.SemaphoreType.DMA((NBUF,)),
            pltpu.SemaphoreType.DMA((NBUF,)),
        ],
        compiler_params=pltpu.CompilerParams(use_tc_tiling_on_sc=False),
    )(_gather_body)
    idx2d = x_flat.reshape(B_TOTAL // CHUNK, CHUNK)
    return k(embeddings, idx2d)


def kernel(x, embeddings):
    out = _embedding_gather(x.reshape(-1), embeddings)
    return out.reshape(BATCH, FIELDS, EMBEDDING_DIM)
